# baseline (speedup 1.0000x reference)
import jax
import jax.numpy as jnp
from jax import lax
from jax.experimental import pallas as pl
from jax.experimental.pallas import tpu as pltpu
from jax.experimental.pallas import tpu_sc as plsc

VOCAB = 100000
DIM = 64
BATCH = 16384

NUM_CORES = 2
NUM_SUBCORES = 16
LANES = 16
NW = NUM_CORES * NUM_SUBCORES
BPW = BATCH // NW
CH = 64
NCHUNK = BPW // CH
NBUF = 2

BIAS_ROWS = (VOCAB + 127) // 128
BIAS_PAD = BIAS_ROWS * 128 - VOCAB

HALF = VOCAB // 2
REPACK_BLK = 2000


def _repack_body(t_lo, t_hi, c_lo, c_hi, t_out, c_out):
    t_out[...] = jnp.concatenate([t_lo[...], t_hi[...]], axis=1)
    c_out[...] = jnp.concatenate([c_lo[...], c_hi[...]], axis=1)


def _repack(temb, cemb):
    grid = HALF // REPACK_BLK
    nlo = grid
    lo_spec = pl.BlockSpec((REPACK_BLK, DIM), lambda i: (i, 0))
    hi_spec = pl.BlockSpec((REPACK_BLK, DIM), lambda i: (i + nlo, 0))
    out_spec = pl.BlockSpec((REPACK_BLK, 128), lambda i: (i, 0))
    return pl.pallas_call(
        _repack_body,
        grid=(grid,),
        in_specs=[lo_spec, hi_spec, lo_spec, hi_spec],
        out_specs=[out_spec, out_spec],
        out_shape=[
            jax.ShapeDtypeStruct((HALF, 128), jnp.float32),
            jax.ShapeDtypeStruct((HALF, 128), jnp.float32),
        ],
    )(temb, temb, cemb, cemb)


def _glove_body(tix_hbm, cix_hbm, t2_hbm, c2_hbm, tbp_hbm, cbp_hbm,
                out_hbm, idx_t, idx_c, idx_te, idx_ce, idx_tb, idx_cb,
                te, ce, tbr, cbr, outv, sems):
    wid = lax.axis_index("s") * NUM_CORES + lax.axis_index("c")
    base = wid * BPW

    pltpu.sync_copy(tix_hbm.at[pl.ds(base, BPW)], idx_t)
    pltpu.sync_copy(cix_hbm.at[pl.ds(base, BPW)], idx_c)

    def shift_body(g, carry):
        j0 = g * LANES
        ti = idx_t[pl.ds(j0, LANES)]
        ci = idx_c[pl.ds(j0, LANES)]
        idx_te[pl.ds(j0, LANES)] = jnp.where(ti >= HALF, ti - HALF, ti)
        idx_ce[pl.ds(j0, LANES)] = jnp.where(ci >= HALF, ci - HALF, ci)
        idx_tb[pl.ds(j0, LANES)] = lax.shift_right_logical(ti, 7)
        idx_cb[pl.ds(j0, LANES)] = lax.shift_right_logical(ci, 7)
        return carry

    lax.fori_loop(0, BPW // LANES, shift_body, 0)

    def fire(c0, b):
        pltpu.async_copy(t2_hbm.at[idx_te.at[pl.ds(c0, CH)]], te.at[b], sems.at[b, 0])
        pltpu.async_copy(c2_hbm.at[idx_ce.at[pl.ds(c0, CH)]], ce.at[b], sems.at[b, 1])
        pltpu.async_copy(tbp_hbm.at[idx_tb.at[pl.ds(c0, CH)]], tbr.at[b], sems.at[b, 2])
        pltpu.async_copy(cbp_hbm.at[idx_cb.at[pl.ds(c0, CH)]], cbr.at[b], sems.at[b, 3])

    def drain(c0, b):
        pltpu.make_async_copy(t2_hbm.at[idx_te.at[pl.ds(c0, CH)]], te.at[b], sems.at[b, 0]).wait()
        pltpu.make_async_copy(c2_hbm.at[idx_ce.at[pl.ds(c0, CH)]], ce.at[b], sems.at[b, 1]).wait()
        pltpu.make_async_copy(tbp_hbm.at[idx_tb.at[pl.ds(c0, CH)]], tbr.at[b], sems.at[b, 2]).wait()
        pltpu.make_async_copy(cbp_hbm.at[idx_cb.at[pl.ds(c0, CH)]], cbr.at[b], sems.at[b, 3]).wait()

    lane = lax.iota(jnp.int32, LANES)
    dn = lax.GatherDimensionNumbers(
        offset_dims=(), collapsed_slice_dims=(0,), start_index_map=(0,))
    perms = [(lane ^ sh).reshape(LANES, 1) for sh in (1, 2, 4, 8)]

    def hsum(v):
        for p_ix in perms:
            v = v + lax.gather(v, p_ix, dn, slice_sizes=(1,),
                               mode=lax.GatherScatterMode.PROMISE_IN_BOUNDS)
        return v

    def compute_chunk(c0, b):
        bvec = jnp.full((LANES,), b, jnp.int32)
        for g in range(CH // LANES):
            j0 = g * LANES
            rowv = j0 + lane
            tiv = idx_t[pl.ds(c0 + j0, LANES)]
            civ = idx_c[pl.ds(c0 + j0, LANES)]
            tb_sel = plsc.load_gather(tbr, [bvec, rowv, tiv & 127])
            cb_sel = plsc.load_gather(cbr, [bvec, rowv, civ & 127])
            acc = tb_sel + cb_sel
            hta = jnp.where(tiv >= HALF, DIM, 0)
            hca = jnp.where(civ >= HALF, DIM, 0)
            for r in range(LANES):
                ht = hta[r]
                hc = hca[r]
                p = te[b, j0 + r, pl.ds(ht, LANES)] * ce[b, j0 + r, pl.ds(hc, LANES)]
                for k in range(1, DIM // LANES):
                    p = p + (te[b, j0 + r, pl.ds(ht + k * LANES, LANES)]
                             * ce[b, j0 + r, pl.ds(hc + k * LANES, LANES)])
                acc = jnp.where(lane == r, hsum(p) + acc, acc)
            outv[pl.ds(c0 + j0, LANES)] = acc

    last = (NCHUNK - 1) * CH

    for b in range(NBUF):
        fire(b * CH, b)

    def step(g, carry):
        for b in range(NBUF):
            c0 = (NBUF * g + b) * CH
            drain(c0, b)
            compute_chunk(c0, b)
            nxt = jnp.minimum(c0 + NBUF * CH, last)
            fire(nxt, b)
        return carry

    lax.fori_loop(0, NCHUNK // NBUF, step, 0)

    for b in range(NBUF):
        drain(last, b)

    pltpu.sync_copy(outv, out_hbm.at[pl.ds(base, BPW)])


@jax.jit
def kernel(inputs, target_emb, target_bias, context_emb, context_bias):
    t_ix = inputs[:, 0].astype(jnp.int32)
    c_ix = inputs[:, 1].astype(jnp.int32)
    t2, c2 = _repack(target_emb, context_emb)
    tbp = jnp.pad(target_bias.reshape(VOCAB), (0, BIAS_PAD)).reshape(BIAS_ROWS, 128)
    cbp = jnp.pad(context_bias.reshape(VOCAB), (0, BIAS_PAD)).reshape(BIAS_ROWS, 128)

    mesh = plsc.VectorSubcoreMesh(
        core_axis_name="c", subcore_axis_name="s",
        num_cores=NUM_CORES, num_subcores=NUM_SUBCORES)

    run = pl.kernel(
        _glove_body,
        out_type=jax.ShapeDtypeStruct((BATCH,), jnp.float32),
        mesh=mesh,
        compiler_params=pltpu.CompilerParams(
            use_tc_tiling_on_sc=True, needs_layout_passes=False),
        scratch_types=[
            pltpu.VMEM((BPW,), jnp.int32),
            pltpu.VMEM((BPW,), jnp.int32),
            pltpu.VMEM((BPW,), jnp.int32),
            pltpu.VMEM((BPW,), jnp.int32),
            pltpu.VMEM((BPW,), jnp.int32),
            pltpu.VMEM((BPW,), jnp.int32),
            pltpu.VMEM((NBUF, CH, 128), jnp.float32),
            pltpu.VMEM((NBUF, CH, 128), jnp.float32),
            pltpu.VMEM((NBUF, CH, 128), jnp.float32),
            pltpu.VMEM((NBUF, CH, 128), jnp.float32),
            pltpu.VMEM((BPW,), jnp.float32),
            pltpu.SemaphoreType.DMA((NBUF, 4)),
        ],
    )
    out = run(t_ix, c_ix, t2, c2, tbp, cbp)
    return out.reshape(BATCH, 1)


# speedup vs baseline: 1.0682x; 1.0682x over previous
import jax
import jax.numpy as jnp
from jax import lax
from jax.experimental import pallas as pl
from jax.experimental.pallas import tpu as pltpu
from jax.experimental.pallas import tpu_sc as plsc

VOCAB = 100000
DIM = 64
BATCH = 16384

NUM_CORES = 2
NUM_SUBCORES = 16
LANES = 16
NW = NUM_CORES * NUM_SUBCORES
BPW = BATCH // NW
CH = 64
NCHUNK = BPW // CH
NBUF = 2

BIAS_ROWS = (VOCAB + 127) // 128
BIAS_PAD = BIAS_ROWS * 128 - VOCAB

HALF = VOCAB // 2


def _glove_body(tix_hbm, cix_hbm, t2_hbm, c2_hbm, tbp_hbm, cbp_hbm,
                out_hbm, idx_t, idx_c, idx_te, idx_ce, idx_tb, idx_cb,
                te, ce, tbr, cbr, outv, sems):
    wid = lax.axis_index("s") * NUM_CORES + lax.axis_index("c")
    base = wid * BPW

    pltpu.sync_copy(tix_hbm.at[pl.ds(base, BPW)], idx_t)
    pltpu.sync_copy(cix_hbm.at[pl.ds(base, BPW)], idx_c)

    def shift_body(g, carry):
        j0 = g * LANES
        ti = idx_t[pl.ds(j0, LANES)]
        ci = idx_c[pl.ds(j0, LANES)]
        idx_te[pl.ds(j0, LANES)] = lax.shift_right_logical(ti, 1)
        idx_ce[pl.ds(j0, LANES)] = lax.shift_right_logical(ci, 1)
        idx_tb[pl.ds(j0, LANES)] = lax.shift_right_logical(ti, 7)
        idx_cb[pl.ds(j0, LANES)] = lax.shift_right_logical(ci, 7)
        return carry

    lax.fori_loop(0, BPW // LANES, shift_body, 0)

    def fire(c0, b):
        pltpu.async_copy(t2_hbm.at[idx_te.at[pl.ds(c0, CH)]], te.at[b], sems.at[b, 0])
        pltpu.async_copy(c2_hbm.at[idx_ce.at[pl.ds(c0, CH)]], ce.at[b], sems.at[b, 1])
        pltpu.async_copy(tbp_hbm.at[idx_tb.at[pl.ds(c0, CH)]], tbr.at[b], sems.at[b, 2])
        pltpu.async_copy(cbp_hbm.at[idx_cb.at[pl.ds(c0, CH)]], cbr.at[b], sems.at[b, 3])

    def drain(c0, b):
        pltpu.make_async_copy(t2_hbm.at[idx_te.at[pl.ds(c0, CH)]], te.at[b], sems.at[b, 0]).wait()
        pltpu.make_async_copy(c2_hbm.at[idx_ce.at[pl.ds(c0, CH)]], ce.at[b], sems.at[b, 1]).wait()
        pltpu.make_async_copy(tbp_hbm.at[idx_tb.at[pl.ds(c0, CH)]], tbr.at[b], sems.at[b, 2]).wait()
        pltpu.make_async_copy(cbp_hbm.at[idx_cb.at[pl.ds(c0, CH)]], cbr.at[b], sems.at[b, 3]).wait()

    lane = lax.iota(jnp.int32, LANES)
    dn = lax.GatherDimensionNumbers(
        offset_dims=(), collapsed_slice_dims=(0,), start_index_map=(0,))
    perms = [(lane ^ sh).reshape(LANES, 1) for sh in (1, 2, 4, 8)]

    def hsum(v):
        for p_ix in perms:
            v = v + lax.gather(v, p_ix, dn, slice_sizes=(1,),
                               mode=lax.GatherScatterMode.PROMISE_IN_BOUNDS)
        return v

    def compute_chunk(c0, b):
        bvec = jnp.full((LANES,), b, jnp.int32)
        for g in range(CH // LANES):
            j0 = g * LANES
            rowv = j0 + lane
            tiv = idx_t[pl.ds(c0 + j0, LANES)]
            civ = idx_c[pl.ds(c0 + j0, LANES)]
            tb_sel = plsc.load_gather(tbr, [bvec, rowv, tiv & 127])
            cb_sel = plsc.load_gather(cbr, [bvec, rowv, civ & 127])
            acc = tb_sel + cb_sel
            hta = (tiv & 1) * DIM
            hca = (civ & 1) * DIM
            for r in range(LANES):
                ht = hta[r]
                hc = hca[r]
                p = te[b, j0 + r, pl.ds(ht, LANES)] * ce[b, j0 + r, pl.ds(hc, LANES)]
                for k in range(1, DIM // LANES):
                    p = p + (te[b, j0 + r, pl.ds(ht + k * LANES, LANES)]
                             * ce[b, j0 + r, pl.ds(hc + k * LANES, LANES)])
                acc = jnp.where(lane == r, hsum(p) + acc, acc)
            outv[pl.ds(c0 + j0, LANES)] = acc

    last = (NCHUNK - 1) * CH

    for b in range(NBUF):
        fire(b * CH, b)

    def step(g, carry):
        for b in range(NBUF):
            c0 = (NBUF * g + b) * CH
            drain(c0, b)
            compute_chunk(c0, b)
            nxt = jnp.minimum(c0 + NBUF * CH, last)
            fire(nxt, b)
        return carry

    lax.fori_loop(0, NCHUNK // NBUF, step, 0)

    for b in range(NBUF):
        drain(last, b)

    pltpu.sync_copy(outv, out_hbm.at[pl.ds(base, BPW)])


@jax.jit
def kernel(inputs, target_emb, target_bias, context_emb, context_bias):
    t_ix = inputs[:, 0].astype(jnp.int32)
    c_ix = inputs[:, 1].astype(jnp.int32)
    t2 = target_emb.reshape(HALF, 128)
    c2 = context_emb.reshape(HALF, 128)
    tbp = jnp.pad(target_bias.reshape(VOCAB), (0, BIAS_PAD)).reshape(BIAS_ROWS, 128)
    cbp = jnp.pad(context_bias.reshape(VOCAB), (0, BIAS_PAD)).reshape(BIAS_ROWS, 128)

    mesh = plsc.VectorSubcoreMesh(
        core_axis_name="c", subcore_axis_name="s",
        num_cores=NUM_CORES, num_subcores=NUM_SUBCORES)

    run = pl.kernel(
        _glove_body,
        out_type=jax.ShapeDtypeStruct((BATCH,), jnp.float32),
        mesh=mesh,
        compiler_params=pltpu.CompilerParams(
            use_tc_tiling_on_sc=True, needs_layout_passes=False),
        scratch_types=[
            pltpu.VMEM((BPW,), jnp.int32),
            pltpu.VMEM((BPW,), jnp.int32),
            pltpu.VMEM((BPW,), jnp.int32),
            pltpu.VMEM((BPW,), jnp.int32),
            pltpu.VMEM((BPW,), jnp.int32),
            pltpu.VMEM((BPW,), jnp.int32),
            pltpu.VMEM((NBUF, CH, 128), jnp.float32),
            pltpu.VMEM((NBUF, CH, 128), jnp.float32),
            pltpu.VMEM((NBUF, CH, 128), jnp.float32),
            pltpu.VMEM((NBUF, CH, 128), jnp.float32),
            pltpu.VMEM((BPW,), jnp.float32),
            pltpu.SemaphoreType.DMA((NBUF, 4)),
        ],
    )
    out = run(t_ix, c_ix, t2, c2, tbp, cbp)
    return out.reshape(BATCH, 1)


# speedup vs baseline: 1.0964x; 1.0264x over previous
import jax
import jax.numpy as jnp
from jax import lax
from jax.experimental import pallas as pl
from jax.experimental.pallas import tpu as pltpu
from jax.experimental.pallas import tpu_sc as plsc

VOCAB = 100000
DIM = 64
BATCH = 16384

NUM_CORES = 2
NUM_SUBCORES = 16
LANES = 16
NW = NUM_CORES * NUM_SUBCORES
BPW = BATCH // NW
CH = 64
NCHUNK = BPW // CH
NBUF = 2

BIAS_ROWS = (VOCAB + 127) // 128
BIAS_PAD = BIAS_ROWS * 128 - VOCAB

HALF = VOCAB // 2


def _glove_body(tix_hbm, cix_hbm, t2_hbm, c2_hbm, tbp_hbm, cbp_hbm,
                out_hbm, idx_t, idx_c, idx_tb, idx_cb,
                te, ce, tbr, cbr, outv, sems):
    wid = lax.axis_index("s") * NUM_CORES + lax.axis_index("c")
    base = wid * BPW

    pltpu.sync_copy(tix_hbm.at[pl.ds(base, BPW)], idx_t)
    pltpu.sync_copy(cix_hbm.at[pl.ds(base, BPW)], idx_c)

    def shift_body(g, carry):
        j0 = g * LANES
        ti = idx_t[pl.ds(j0, LANES)]
        ci = idx_c[pl.ds(j0, LANES)]
        idx_tb[pl.ds(j0, LANES)] = lax.shift_right_logical(ti, 7)
        idx_cb[pl.ds(j0, LANES)] = lax.shift_right_logical(ci, 7)
        return carry

    lax.fori_loop(0, BPW // LANES, shift_body, 0)

    def fire(c0, b):
        pltpu.async_copy(t2_hbm.at[idx_t.at[pl.ds(c0, CH)]], te.at[b], sems.at[b, 0])
        pltpu.async_copy(c2_hbm.at[idx_c.at[pl.ds(c0, CH)]], ce.at[b], sems.at[b, 1])
        pltpu.async_copy(tbp_hbm.at[idx_tb.at[pl.ds(c0, CH)]], tbr.at[b], sems.at[b, 2])
        pltpu.async_copy(cbp_hbm.at[idx_cb.at[pl.ds(c0, CH)]], cbr.at[b], sems.at[b, 3])

    def drain(c0, b):
        pltpu.make_async_copy(t2_hbm.at[idx_t.at[pl.ds(c0, CH)]], te.at[b], sems.at[b, 0]).wait()
        pltpu.make_async_copy(c2_hbm.at[idx_c.at[pl.ds(c0, CH)]], ce.at[b], sems.at[b, 1]).wait()
        pltpu.make_async_copy(tbp_hbm.at[idx_tb.at[pl.ds(c0, CH)]], tbr.at[b], sems.at[b, 2]).wait()
        pltpu.make_async_copy(cbp_hbm.at[idx_cb.at[pl.ds(c0, CH)]], cbr.at[b], sems.at[b, 3]).wait()

    lane = lax.iota(jnp.int32, LANES)
    dn = lax.GatherDimensionNumbers(
        offset_dims=(), collapsed_slice_dims=(0,), start_index_map=(0,))
    perms = [(lane ^ sh).reshape(LANES, 1) for sh in (1, 2, 4, 8)]

    def hsum(v):
        for p_ix in perms:
            v = v + lax.gather(v, p_ix, dn, slice_sizes=(1,),
                               mode=lax.GatherScatterMode.PROMISE_IN_BOUNDS)
        return v

    def compute_chunk(c0, b):
        bvec = jnp.full((LANES,), b, jnp.int32)
        for g in range(CH // LANES):
            j0 = g * LANES
            rowv = j0 + lane
            tiv = idx_t[pl.ds(c0 + j0, LANES)]
            civ = idx_c[pl.ds(c0 + j0, LANES)]
            tb_sel = plsc.load_gather(tbr, [bvec, rowv, tiv & 127])
            cb_sel = plsc.load_gather(cbr, [bvec, rowv, civ & 127])
            acc = tb_sel + cb_sel
            for r in range(LANES):
                p = te[b, j0 + r, pl.ds(0, LANES)] * ce[b, j0 + r, pl.ds(0, LANES)]
                for k in range(1, DIM // LANES):
                    p = p + (te[b, j0 + r, pl.ds(k * LANES, LANES)]
                             * ce[b, j0 + r, pl.ds(k * LANES, LANES)])
                acc = jnp.where(lane == r, hsum(p) + acc, acc)
            outv[pl.ds(c0 + j0, LANES)] = acc

    last = (NCHUNK - 1) * CH

    for b in range(NBUF):
        fire(b * CH, b)

    def step(g, carry):
        for b in range(NBUF):
            c0 = (NBUF * g + b) * CH
            drain(c0, b)
            compute_chunk(c0, b)
            nxt = jnp.minimum(c0 + NBUF * CH, last)
            fire(nxt, b)
        return carry

    lax.fori_loop(0, NCHUNK // NBUF, step, 0)

    for b in range(NBUF):
        drain(last, b)

    pltpu.sync_copy(outv, out_hbm.at[pl.ds(base, BPW)])


@jax.jit
def kernel(inputs, target_emb, target_bias, context_emb, context_bias):
    t_ix = inputs[:, 0].astype(jnp.int32)
    c_ix = inputs[:, 1].astype(jnp.int32)
    t2 = target_emb
    c2 = context_emb
    tbp = jnp.pad(target_bias.reshape(VOCAB), (0, BIAS_PAD)).reshape(BIAS_ROWS, 128)
    cbp = jnp.pad(context_bias.reshape(VOCAB), (0, BIAS_PAD)).reshape(BIAS_ROWS, 128)

    mesh = plsc.VectorSubcoreMesh(
        core_axis_name="c", subcore_axis_name="s",
        num_cores=NUM_CORES, num_subcores=NUM_SUBCORES)

    run = pl.kernel(
        _glove_body,
        out_type=jax.ShapeDtypeStruct((BATCH,), jnp.float32),
        mesh=mesh,
        compiler_params=pltpu.CompilerParams(
            use_tc_tiling_on_sc=False, needs_layout_passes=False),
        scratch_types=[
            pltpu.VMEM((BPW,), jnp.int32),
            pltpu.VMEM((BPW,), jnp.int32),
            pltpu.VMEM((BPW,), jnp.int32),
            pltpu.VMEM((BPW,), jnp.int32),
            pltpu.VMEM((NBUF, CH, DIM), jnp.float32),
            pltpu.VMEM((NBUF, CH, DIM), jnp.float32),
            pltpu.VMEM((NBUF, CH, 128), jnp.float32),
            pltpu.VMEM((NBUF, CH, 128), jnp.float32),
            pltpu.VMEM((BPW,), jnp.float32),
            pltpu.SemaphoreType.DMA((NBUF, 4)),
        ],
    )
    out = run(t_ix, c_ix, t2, c2, tbp, cbp)
    return out.reshape(BATCH, 1)


# speedup vs baseline: 1.0997x; 1.0030x over previous
import jax
import jax.numpy as jnp
from jax import lax
from jax.experimental import pallas as pl
from jax.experimental.pallas import tpu as pltpu
from jax.experimental.pallas import tpu_sc as plsc

VOCAB = 100000
DIM = 64
BATCH = 16384

NUM_CORES = 2
NUM_SUBCORES = 16
LANES = 16
NW = NUM_CORES * NUM_SUBCORES
BPW = BATCH // NW
QB = 128
QPASS = BPW // QB

BIAS_ROWS = (VOCAB + 127) // 128
BIAS_PAD = BIAS_ROWS * 128 - VOCAB

HALF = VOCAB // 2


def _glove_body(tix_hbm, cix_hbm, t2_hbm, c2_hbm, tbp_hbm, cbp_hbm,
                out_hbm, idx_t, idx_c, idx_tb, idx_cb,
                te, ce, tbr, cbr, bsel, outv, sem0, sem1, sem2, sem3):
    wid = lax.axis_index("s") * NUM_CORES + lax.axis_index("c")
    base = wid * BPW

    pltpu.sync_copy(tix_hbm.at[pl.ds(base, BPW)], idx_t)
    pltpu.sync_copy(cix_hbm.at[pl.ds(base, BPW)], idx_c)

    def shift_body(g, carry):
        j0 = g * LANES
        ti = idx_t[pl.ds(j0, LANES)]
        ci = idx_c[pl.ds(j0, LANES)]
        idx_tb[pl.ds(j0, LANES)] = lax.shift_right_logical(ti, 7)
        idx_cb[pl.ds(j0, LANES)] = lax.shift_right_logical(ci, 7)
        return carry

    lax.fori_loop(0, BPW // LANES, shift_body, 0)

    ge0 = pltpu.async_copy(t2_hbm.at[idx_t], te, sem0)
    ge1 = pltpu.async_copy(c2_hbm.at[idx_c], ce, sem1)

    lane = lax.iota(jnp.int32, LANES)
    dn = lax.GatherDimensionNumbers(
        offset_dims=(), collapsed_slice_dims=(0,), start_index_map=(0,))
    perms = [(lane ^ sh).reshape(LANES, 1) for sh in (1, 2, 4, 8)]

    def hsum(v):
        for p_ix in perms:
            v = v + lax.gather(v, p_ix, dn, slice_sizes=(1,),
                               mode=lax.GatherScatterMode.PROMISE_IN_BOUNDS)
        return v

    for q in range(QPASS):
        c0 = q * QB
        g2 = pltpu.async_copy(tbp_hbm.at[idx_tb.at[pl.ds(c0, QB)]], tbr, sem2)
        g3 = pltpu.async_copy(cbp_hbm.at[idx_cb.at[pl.ds(c0, QB)]], cbr, sem3)
        g2.wait()
        g3.wait()
        for g in range(QB // LANES):
            j0 = g * LANES
            rowv = j0 + lane
            tiv = idx_t[pl.ds(c0 + j0, LANES)]
            civ = idx_c[pl.ds(c0 + j0, LANES)]
            tb_sel = plsc.load_gather(tbr, [rowv, tiv & 127])
            cb_sel = plsc.load_gather(cbr, [rowv, civ & 127])
            bsel[pl.ds(c0 + j0, LANES)] = tb_sel + cb_sel

    ge0.wait()
    ge1.wait()

    def group(g, carry):
        j0 = g * LANES
        acc = bsel[pl.ds(j0, LANES)]
        for r in range(LANES):
            row = j0 + r
            p = te[row, pl.ds(0, LANES)] * ce[row, pl.ds(0, LANES)]
            for k in range(1, DIM // LANES):
                p = p + te[row, pl.ds(k * LANES, LANES)] * ce[row, pl.ds(k * LANES, LANES)]
            acc = jnp.where(lane == r, hsum(p) + acc, acc)
        outv[pl.ds(j0, LANES)] = acc
        return carry

    lax.fori_loop(0, BPW // LANES, group, 0)

    pltpu.sync_copy(outv, out_hbm.at[pl.ds(base, BPW)])


@jax.jit
def kernel(inputs, target_emb, target_bias, context_emb, context_bias):
    t_ix = inputs[:, 0].astype(jnp.int32)
    c_ix = inputs[:, 1].astype(jnp.int32)
    t2 = target_emb
    c2 = context_emb
    tbp = jnp.pad(target_bias.reshape(VOCAB), (0, BIAS_PAD)).reshape(BIAS_ROWS, 128)
    cbp = jnp.pad(context_bias.reshape(VOCAB), (0, BIAS_PAD)).reshape(BIAS_ROWS, 128)

    mesh = plsc.VectorSubcoreMesh(
        core_axis_name="c", subcore_axis_name="s",
        num_cores=NUM_CORES, num_subcores=NUM_SUBCORES)

    run = pl.kernel(
        _glove_body,
        out_type=jax.ShapeDtypeStruct((BATCH,), jnp.float32),
        mesh=mesh,
        compiler_params=pltpu.CompilerParams(
            use_tc_tiling_on_sc=False, needs_layout_passes=False),
        scratch_types=[
            pltpu.VMEM((BPW,), jnp.int32),
            pltpu.VMEM((BPW,), jnp.int32),
            pltpu.VMEM((BPW,), jnp.int32),
            pltpu.VMEM((BPW,), jnp.int32),
            pltpu.VMEM((BPW, DIM), jnp.float32),
            pltpu.VMEM((BPW, DIM), jnp.float32),
            pltpu.VMEM((QB, 128), jnp.float32),
            pltpu.VMEM((QB, 128), jnp.float32),
            pltpu.VMEM((BPW,), jnp.float32),
            pltpu.VMEM((BPW,), jnp.float32),
            pltpu.SemaphoreType.DMA,
            pltpu.SemaphoreType.DMA,
            pltpu.SemaphoreType.DMA,
            pltpu.SemaphoreType.DMA,
        ],
    )
    out = run(t_ix, c_ix, t2, c2, tbp, cbp)
    return out.reshape(BATCH, 1)


# speedup vs baseline: 1.1551x; 1.0504x over previous
import jax
import jax.numpy as jnp
from jax import lax
from jax.experimental import pallas as pl
from jax.experimental.pallas import tpu as pltpu
from jax.experimental.pallas import tpu_sc as plsc

VOCAB = 100000
DIM = 64
BATCH = 16384

NUM_CORES = 2
NUM_SUBCORES = 16
LANES = 16
NW = NUM_CORES * NUM_SUBCORES
BPW = BATCH // NW
QB = 128
QPASS = BPW // QB

BIAS_ROWS = VOCAB // LANES

HALF = VOCAB // 2


def _glove_body(tix_hbm, cix_hbm, t2_hbm, c2_hbm, tbp_hbm, cbp_hbm,
                out_hbm, idx_t, idx_c, idx_tb, idx_cb,
                te, ce, tbr, cbr, bsel, outv, sem0, sem1, sem2, sem3):
    wid = lax.axis_index("s") * NUM_CORES + lax.axis_index("c")
    base = wid * BPW

    pltpu.sync_copy(tix_hbm.at[pl.ds(base, BPW)], idx_t)
    pltpu.sync_copy(cix_hbm.at[pl.ds(base, BPW)], idx_c)

    def shift_body(g, carry):
        j0 = g * LANES
        ti = idx_t[pl.ds(j0, LANES)]
        ci = idx_c[pl.ds(j0, LANES)]
        idx_tb[pl.ds(j0, LANES)] = lax.shift_right_logical(ti, 4)
        idx_cb[pl.ds(j0, LANES)] = lax.shift_right_logical(ci, 4)
        return carry

    lax.fori_loop(0, BPW // LANES, shift_body, 0)

    ge0 = pltpu.async_copy(t2_hbm.at[idx_t], te, sem0)
    ge1 = pltpu.async_copy(c2_hbm.at[idx_c], ce, sem1)

    lane = lax.iota(jnp.int32, LANES)
    dn = lax.GatherDimensionNumbers(
        offset_dims=(), collapsed_slice_dims=(0,), start_index_map=(0,))
    perms = [(lane ^ sh).reshape(LANES, 1) for sh in (1, 2, 4, 8)]

    def hsum(v):
        for p_ix in perms:
            v = v + lax.gather(v, p_ix, dn, slice_sizes=(1,),
                               mode=lax.GatherScatterMode.PROMISE_IN_BOUNDS)
        return v

    for q in range(QPASS):
        c0 = q * QB
        g2 = pltpu.async_copy(tbp_hbm.at[idx_tb.at[pl.ds(c0, QB)]], tbr, sem2)
        g3 = pltpu.async_copy(cbp_hbm.at[idx_cb.at[pl.ds(c0, QB)]], cbr, sem3)
        g2.wait()
        g3.wait()
        for g in range(QB // LANES):
            j0 = g * LANES
            rowv = j0 + lane
            tiv = idx_t[pl.ds(c0 + j0, LANES)]
            civ = idx_c[pl.ds(c0 + j0, LANES)]
            tb_sel = plsc.load_gather(tbr, [rowv, tiv & (LANES - 1)])
            cb_sel = plsc.load_gather(cbr, [rowv, civ & (LANES - 1)])
            bsel[pl.ds(c0 + j0, LANES)] = tb_sel + cb_sel

    ge0.wait()
    ge1.wait()

    def group(g, carry):
        j0 = g * LANES
        acc = bsel[pl.ds(j0, LANES)]
        for r in range(LANES):
            row = j0 + r
            p = te[row, pl.ds(0, LANES)] * ce[row, pl.ds(0, LANES)]
            for k in range(1, DIM // LANES):
                p = p + te[row, pl.ds(k * LANES, LANES)] * ce[row, pl.ds(k * LANES, LANES)]
            acc = jnp.where(lane == r, hsum(p) + acc, acc)
        outv[pl.ds(j0, LANES)] = acc
        return carry

    lax.fori_loop(0, BPW // LANES, group, 0)

    pltpu.sync_copy(outv, out_hbm.at[pl.ds(base, BPW)])


@jax.jit
def kernel(inputs, target_emb, target_bias, context_emb, context_bias):
    t_ix = inputs[:, 0].astype(jnp.int32)
    c_ix = inputs[:, 1].astype(jnp.int32)
    t2 = target_emb
    c2 = context_emb
    tbp = target_bias.reshape(BIAS_ROWS, LANES)
    cbp = context_bias.reshape(BIAS_ROWS, LANES)

    mesh = plsc.VectorSubcoreMesh(
        core_axis_name="c", subcore_axis_name="s",
        num_cores=NUM_CORES, num_subcores=NUM_SUBCORES)

    run = pl.kernel(
        _glove_body,
        out_type=jax.ShapeDtypeStruct((BATCH,), jnp.float32),
        mesh=mesh,
        compiler_params=pltpu.CompilerParams(
            use_tc_tiling_on_sc=False, needs_layout_passes=False),
        scratch_types=[
            pltpu.VMEM((BPW,), jnp.int32),
            pltpu.VMEM((BPW,), jnp.int32),
            pltpu.VMEM((BPW,), jnp.int32),
            pltpu.VMEM((BPW,), jnp.int32),
            pltpu.VMEM((BPW, DIM), jnp.float32),
            pltpu.VMEM((BPW, DIM), jnp.float32),
            pltpu.VMEM((QB, LANES), jnp.float32),
            pltpu.VMEM((QB, LANES), jnp.float32),
            pltpu.VMEM((BPW,), jnp.float32),
            pltpu.VMEM((BPW,), jnp.float32),
            pltpu.SemaphoreType.DMA,
            pltpu.SemaphoreType.DMA,
            pltpu.SemaphoreType.DMA,
            pltpu.SemaphoreType.DMA,
        ],
    )
    out = run(t_ix, c_ix, t2, c2, tbp, cbp)
    return out.reshape(BATCH, 1)
